# baseline (speedup 1.0000x reference)
import functools

import jax
import jax.numpy as jnp
from jax.experimental import pallas as pl
from jax.experimental.pallas import tpu as pltpu

STRICT_BONUS = 10.0
NOISE_EPS = 1e-07


def _shift_w(v, dw):
    if dw == -1:
        return jnp.concatenate([v[..., :, :1], v[..., :, :-1]], axis=-1)
    return jnp.concatenate([v[..., :, 1:], v[..., :, -1:]], axis=-1)


def _stencil_kernel(x_ref, noise_ref, coords_ref, y_ref, win_ref, *, BC, D, H, W, TH, T):
    t = pl.program_id(0)
    r0 = t * TH

    row_f = (jax.lax.broadcasted_iota(jnp.int32, (BC, TH, W), 1) + r0).astype(jnp.float32)
    col_f = jax.lax.broadcasted_iota(jnp.int32, (BC, TH, W), 2).astype(jnp.float32)

    if T == 1:
        for b in range(BC):
            for z in range(D):
                win_ref[b, z, 8 : TH + 8] = x_ref[b, z]
                win_ref[b, z, 7:8] = x_ref[b, z, :1]
                win_ref[b, z, TH + 8 : TH + 9] = x_ref[b, z, H - 1 :]
    else:
        @pl.when(t == 0)
        def _first():
            for b in range(BC):
                for z in range(D):
                    win_ref[b, z, 8:] = x_ref[b, z, : TH + 8]
                    win_ref[b, z, 7:8] = x_ref[b, z, :1]

        @pl.when((t > 0) & (t < T - 1))
        def _middle():
            for b in range(BC):
                for z in range(D):
                    win_ref[b, z] = x_ref[b, z, pl.ds(r0 - 8, TH + 16)]

        @pl.when(t == T - 1)
        def _last():
            for b in range(BC):
                for z in range(D):
                    win_ref[b, z, : TH + 8] = x_ref[b, z, pl.ds(H - TH - 8, TH + 8)]
                    win_ref[b, z, TH + 8 : TH + 9] = x_ref[b, z, H - 1 :]

    for d in (0, D - 1):
        y_ref[:, 0, d] = win_ref[:, d, 8 : TH + 8]
        coords_ref[:, 0, 0, d] = jnp.full((BC, TH, W), float(d), jnp.float32)
        coords_ref[:, 0, 1, d] = row_f
        coords_ref[:, 0, 2, d] = col_f

    for d in range(1, D - 1):
        n00 = noise_ref[0, 0]; n01 = noise_ref[0, 1]; n02 = noise_ref[0, 2]
        n10 = noise_ref[1, 0]; n11 = noise_ref[1, 1]; n12 = noise_ref[1, 2]
        n20 = noise_ref[2, 0]; n21 = noise_ref[2, 1]; n22 = noise_ref[2, 2]

        cw = win_ref[:, d, 7 : TH + 9]
        low = win_ref[:, d - 1, 7 : TH + 9]
        hiw = win_ref[:, d + 1, 7 : TH + 9]

        z0 = cw[:, 1 : TH + 1]
        c_hm = cw[:, :TH]
        c_hp = cw[:, 2 : TH + 2]
        zlo = low[:, 1 : TH + 1]
        zhi = hiw[:, 1 : TH + 1]
        c_wm = _shift_w(z0, -1); c_wp = _shift_w(z0, 1)

        gx = 0.5 * (c_wp - c_wm)
        gy = 0.5 * (c_hp - c_hm)
        dxx = c_wp + c_wm - 2.0 * z0
        dyy = c_hp + c_hm - 2.0 * z0

        uw = hiw - low
        u = uw[:, 1 : TH + 1]
        gs = 0.5 * u
        dss = zhi + zlo - 2.0 * z0
        dys = 0.25 * (uw[:, 2 : TH + 2] - uw[:, :TH])
        dxs = 0.25 * (_shift_w(u, 1) - _shift_w(u, -1))
        tt = c_hp - c_hm
        dxy = 0.25 * (_shift_w(tt, 1) - _shift_w(tt, -1))

        vm_c = jnp.maximum(jnp.maximum(c_hm, c_hp), z0)
        max8 = jnp.maximum(jnp.maximum(_shift_w(vm_c, -1), _shift_w(vm_c, 1)),
                           jnp.maximum(c_hm, c_hp))
        pmw = jnp.maximum(low, hiw)
        vm_pm = jnp.maximum(jnp.maximum(pmw[:, :TH], pmw[:, 2 : TH + 2]),
                            pmw[:, 1 : TH + 1])
        max9z = jnp.maximum(jnp.maximum(_shift_w(vm_pm, -1), _shift_w(vm_pm, 1)),
                            vm_pm)
        mask = z0 > jnp.maximum(max8, max9z)

        ha = dxx + n00; hb = dxy + n01; hc = dxs + n02
        hd = dxy + n10; he = dyy + n11; hf = dys + n12
        hg = dxs + n20; hh = dys + n21; hi_ = dss + n22

        A11 = he * hi_ - hf * hh; A12 = hc * hh - hb * hi_; A13 = hb * hf - hc * he
        A21 = hf * hg - hd * hi_; A22 = ha * hi_ - hc * hg; A23 = hc * hd - ha * hf
        A31 = hd * hh - he * hg; A32 = hb * hg - ha * hh; A33 = ha * he - hb * hd
        det = ha * A11 + hb * A21 + hc * A31
        neg_inv_det = -1.0 / det
        dx0 = (A11 * gx + A12 * gy + A13 * gs) * neg_inv_det
        dx1 = (A21 * gx + A22 * gy + A23 * gs) * neg_inv_det
        dx2 = (A31 * gx + A32 * gy + A33 * gs) * neg_inv_det

        big = jnp.maximum(jnp.maximum(jnp.abs(dx0), jnp.abs(dx1)),
                          jnp.abs(dx2)) > 0.7
        keep = mask & jnp.logical_not(big)
        dx0 = jnp.where(keep, dx0, 0.0)
        dx1 = jnp.where(keep, dx1, 0.0)
        dx2 = jnp.where(keep, dx2, 0.0)

        dy_corr = 0.5 * (gx * dx0 + gy * dx1 + gs * dx2)
        y_ref[:, 0, d] = z0 + dy_corr + jnp.where(mask, STRICT_BONUS, 0.0)

        coords_ref[:, 0, 0, d] = float(d) + dx2
        coords_ref[:, 0, 1, d] = row_f + dx1
        coords_ref[:, 0, 2, d] = col_f + dx0


@jax.jit
def kernel(x):
    B, C, D, H, W = x.shape
    BC = B * C
    TH = 96 if H % 96 == 0 and H > 96 else H
    T = H // TH
    noise = jnp.abs(jax.random.uniform(jax.random.key(42), (3, 3), dtype=x.dtype)) * NOISE_EPS
    xr = x.reshape(BC, D, H, W)
    coords, y = pl.pallas_call(
        functools.partial(_stencil_kernel, BC=BC, D=D, H=H, W=W, TH=TH, T=T),
        grid=(T,),
        out_shape=(
            jax.ShapeDtypeStruct((B, C, 3, D, H, W), x.dtype),
            jax.ShapeDtypeStruct((B, C, D, H, W), x.dtype),
        ),
        in_specs=[
            pl.BlockSpec((BC, D, H, W), lambda t: (0, 0, 0, 0)),
            pl.BlockSpec(memory_space=pltpu.SMEM),
        ],
        out_specs=(
            pl.BlockSpec((B, C, 3, D, TH, W), lambda t: (0, 0, 0, 0, t, 0)),
            pl.BlockSpec((B, C, D, TH, W), lambda t: (0, 0, 0, t, 0)),
        ),
        scratch_shapes=[pltpu.VMEM((BC, D, TH + 16, W), jnp.float32)],
        compiler_params=pltpu.CompilerParams(
            dimension_semantics=("arbitrary",),
        ),
    )(xr, noise)
    return coords, y


# speedup vs baseline: 1.0106x; 1.0106x over previous
import functools

import jax
import jax.numpy as jnp
from jax.experimental import pallas as pl
from jax.experimental.pallas import tpu as pltpu

STRICT_BONUS = 10.0
NOISE_EPS = 1e-07


def _shift_w(v, dw):
    if dw == -1:
        return jnp.concatenate([v[:, :1], v[:, :-1]], axis=1)
    return jnp.concatenate([v[:, 1:], v[:, -1:]], axis=1)


def _emit_half(x_ref, noise_ref, coords_ref, y_ref, *, D, H, W, TH, r0):
    row_f = (jax.lax.broadcasted_iota(jnp.int32, (TH, W), 0) + r0).astype(jnp.float32)
    col_f = jax.lax.broadcasted_iota(jnp.int32, (TH, W), 1).astype(jnp.float32)

    def window(z):
        lo = x_ref[0, z, max(r0 - 1, 0) : min(r0 + TH + 1, H)]
        parts = []
        if r0 == 0:
            parts.append(x_ref[0, z, :1])
        parts.append(lo)
        if r0 + TH == H:
            parts.append(x_ref[0, z, H - 1 :])
        return jnp.concatenate(parts, axis=0) if len(parts) > 1 else lo

    for d in (0, D - 1):
        y_ref[0, 0, d] = x_ref[0, d, r0 : r0 + TH]
        coords_ref[0, 0, 0, d] = jnp.full((TH, W), float(d), jnp.float32)
        coords_ref[0, 0, 1, d] = row_f
        coords_ref[0, 0, 2, d] = col_f

    for d in range(1, D - 1):
        n00 = noise_ref[0, 0]; n01 = noise_ref[0, 1]; n02 = noise_ref[0, 2]
        n10 = noise_ref[1, 0]; n11 = noise_ref[1, 1]; n12 = noise_ref[1, 2]
        n20 = noise_ref[2, 0]; n21 = noise_ref[2, 1]; n22 = noise_ref[2, 2]

        cw = window(d)
        low = window(d - 1)
        hiw = window(d + 1)

        z0 = cw[1 : TH + 1]
        c_hm = cw[:TH]
        c_hp = cw[2 : TH + 2]
        zlo = low[1 : TH + 1]
        zhi = hiw[1 : TH + 1]
        c_wm = _shift_w(z0, -1); c_wp = _shift_w(z0, 1)

        gx = 0.5 * (c_wp - c_wm)
        gy = 0.5 * (c_hp - c_hm)
        dxx = c_wp + c_wm - 2.0 * z0
        dyy = c_hp + c_hm - 2.0 * z0

        uw = hiw - low
        u = uw[1 : TH + 1]
        gs = 0.5 * u
        dss = zhi + zlo - 2.0 * z0
        dys = 0.25 * (uw[2 : TH + 2] - uw[:TH])
        dxs = 0.25 * (_shift_w(u, 1) - _shift_w(u, -1))
        tt = c_hp - c_hm
        dxy = 0.25 * (_shift_w(tt, 1) - _shift_w(tt, -1))

        vm_c = jnp.maximum(jnp.maximum(c_hm, c_hp), z0)
        max8 = jnp.maximum(jnp.maximum(_shift_w(vm_c, -1), _shift_w(vm_c, 1)),
                           jnp.maximum(c_hm, c_hp))
        pmw = jnp.maximum(low, hiw)
        vm_pm = jnp.maximum(jnp.maximum(pmw[:TH], pmw[2 : TH + 2]),
                            pmw[1 : TH + 1])
        max9z = jnp.maximum(jnp.maximum(_shift_w(vm_pm, -1), _shift_w(vm_pm, 1)),
                            vm_pm)
        mask = z0 > jnp.maximum(max8, max9z)

        ha = dxx + n00; hb = dxy + n01; hc = dxs + n02
        hd = dxy + n10; he = dyy + n11; hf = dys + n12
        hg = dxs + n20; hh = dys + n21; hi_ = dss + n22

        A11 = he * hi_ - hf * hh; A12 = hc * hh - hb * hi_; A13 = hb * hf - hc * he
        A21 = hf * hg - hd * hi_; A22 = ha * hi_ - hc * hg; A23 = hc * hd - ha * hf
        A31 = hd * hh - he * hg; A32 = hb * hg - ha * hh; A33 = ha * he - hb * hd
        det = ha * A11 + hb * A21 + hc * A31
        neg_inv_det = -1.0 / det
        dx0 = (A11 * gx + A12 * gy + A13 * gs) * neg_inv_det
        dx1 = (A21 * gx + A22 * gy + A23 * gs) * neg_inv_det
        dx2 = (A31 * gx + A32 * gy + A33 * gs) * neg_inv_det

        big = jnp.maximum(jnp.maximum(jnp.abs(dx0), jnp.abs(dx1)),
                          jnp.abs(dx2)) > 0.7
        keep = mask & jnp.logical_not(big)
        dx0 = jnp.where(keep, dx0, 0.0)
        dx1 = jnp.where(keep, dx1, 0.0)
        dx2 = jnp.where(keep, dx2, 0.0)

        dy_corr = 0.5 * (gx * dx0 + gy * dx1 + gs * dx2)
        y_ref[0, 0, d] = z0 + dy_corr + jnp.where(mask, STRICT_BONUS, 0.0)

        coords_ref[0, 0, 0, d] = float(d) + dx2
        coords_ref[0, 0, 1, d] = row_f + dx1
        coords_ref[0, 0, 2, d] = col_f + dx0


def _stencil_kernel(x_ref, noise_ref, coords_ref, y_ref, *, D, H, W, TH, T):
    t = pl.program_id(1)
    if T == 1:
        _emit_half(x_ref, noise_ref, coords_ref, y_ref,
                   D=D, H=H, W=W, TH=TH, r0=0)
    else:
        for half in range(T):
            @pl.when(t == half)
            def _half(half=half):
                _emit_half(x_ref, noise_ref, coords_ref, y_ref,
                           D=D, H=H, W=W, TH=TH, r0=half * TH)


@jax.jit
def kernel(x):
    B, C, D, H, W = x.shape
    T = 2 if H % 2 == 0 and H >= 16 else 1
    TH = H // T
    noise = jnp.abs(jax.random.uniform(jax.random.key(42), (3, 3), dtype=x.dtype)) * NOISE_EPS
    xr = x.reshape(B * C, D, H, W)
    coords, y = pl.pallas_call(
        functools.partial(_stencil_kernel, D=D, H=H, W=W, TH=TH, T=T),
        grid=(B * C, T),
        out_shape=(
            jax.ShapeDtypeStruct((B, C, 3, D, H, W), x.dtype),
            jax.ShapeDtypeStruct((B, C, D, H, W), x.dtype),
        ),
        in_specs=[
            pl.BlockSpec((1, D, H, W), lambda b, t: (b, 0, 0, 0)),
            pl.BlockSpec(memory_space=pltpu.SMEM),
        ],
        out_specs=(
            pl.BlockSpec((1, 1, 3, D, TH, W), lambda b, t: (b, 0, 0, 0, t, 0)),
            pl.BlockSpec((1, 1, D, TH, W), lambda b, t: (b, 0, 0, t, 0)),
        ),
        compiler_params=pltpu.CompilerParams(
            dimension_semantics=("arbitrary", "arbitrary"),
        ),
    )(xr, noise)
    return coords, y


# speedup vs baseline: 1.3104x; 1.2966x over previous
import functools

import jax
import jax.numpy as jnp
from jax.experimental import pallas as pl
from jax.experimental.pallas import tpu as pltpu

STRICT_BONUS = 10.0
NOISE_EPS = 1e-07


def _shift_h(v, dh):
    if dh == -1:
        return jnp.concatenate([v[:1, :], v[:-1, :]], axis=0)
    return jnp.concatenate([v[1:, :], v[-1:, :]], axis=0)


def _shift_w(v, dw):
    if dw == -1:
        return jnp.concatenate([v[:, :1], v[:, :-1]], axis=1)
    return jnp.concatenate([v[:, 1:], v[:, -1:]], axis=1)


def _stencil_kernel(x_ref, noise_ref, coords_ref, y_ref, *, D, H, W):
    row_f = jax.lax.broadcasted_iota(jnp.int32, (H, W), 0).astype(jnp.float32)
    col_f = jax.lax.broadcasted_iota(jnp.int32, (H, W), 1).astype(jnp.float32)

    for d in (0, D - 1):
        y_ref[0, 0, d] = x_ref[0, d]
        coords_ref[0, 0, 0, d] = jnp.full((H, W), float(d), jnp.float32)
        coords_ref[0, 0, 1, d] = row_f
        coords_ref[0, 0, 2, d] = col_f

    for d in range(1, D - 1):
        n00 = noise_ref[0, 0]; n01 = noise_ref[0, 1]; n02 = noise_ref[0, 2]
        n10 = noise_ref[1, 0]; n11 = noise_ref[1, 1]; n12 = noise_ref[1, 2]
        n20 = noise_ref[2, 0]; n21 = noise_ref[2, 1]; n22 = noise_ref[2, 2]

        z0 = x_ref[0, d]
        zlo = x_ref[0, d - 1]
        zhi = x_ref[0, d + 1]

        c_hm = _shift_h(z0, -1); c_hp = _shift_h(z0, 1)
        c_wm = _shift_w(z0, -1); c_wp = _shift_w(z0, 1)

        gx = 0.5 * (c_wp - c_wm)
        gy = 0.5 * (c_hp - c_hm)
        dxx = c_wp + c_wm - 2.0 * z0
        dyy = c_hp + c_hm - 2.0 * z0

        u = zhi - zlo
        gs = 0.5 * u
        dss = zhi + zlo - 2.0 * z0
        dys = 0.25 * (_shift_h(u, 1) - _shift_h(u, -1))
        dxs = 0.25 * (_shift_w(u, 1) - _shift_w(u, -1))
        t = c_hp - c_hm
        dxy = 0.25 * (_shift_w(t, 1) - _shift_w(t, -1))

        vm_c = jnp.maximum(jnp.maximum(c_hm, c_hp), z0)
        pm = jnp.maximum(zlo, zhi)
        vm_pm = jnp.maximum(jnp.maximum(_shift_h(pm, -1), _shift_h(pm, 1)), pm)
        q = jnp.maximum(vm_c, vm_pm)
        mx = jnp.maximum(jnp.maximum(_shift_w(q, -1), _shift_w(q, 1)),
                         jnp.maximum(jnp.maximum(c_hm, c_hp), vm_pm))
        mask = z0 > mx

        ha = dxx + n00; hb = dxy + n01; hc = dxs + n02
        hd = dxy + n10; he = dyy + n11; hf = dys + n12
        hg = dxs + n20; hh = dys + n21; hi_ = dss + n22

        A11 = he * hi_ - hf * hh; A12 = hc * hh - hb * hi_; A13 = hb * hf - hc * he
        A21 = hf * hg - hd * hi_; A22 = ha * hi_ - hc * hg; A23 = hc * hd - ha * hf
        A31 = hd * hh - he * hg; A32 = hb * hg - ha * hh; A33 = ha * he - hb * hd
        det = ha * A11 + hb * A21 + hc * A31
        neg_inv_det = -1.0 / det
        dx0 = (A11 * gx + A12 * gy + A13 * gs) * neg_inv_det
        dx1 = (A21 * gx + A22 * gy + A23 * gs) * neg_inv_det
        dx2 = (A31 * gx + A32 * gy + A33 * gs) * neg_inv_det

        big = jnp.maximum(jnp.maximum(jnp.abs(dx0), jnp.abs(dx1)),
                          jnp.abs(dx2)) > 0.7
        keep = mask & jnp.logical_not(big)
        dx0 = jnp.where(keep, dx0, 0.0)
        dx1 = jnp.where(keep, dx1, 0.0)
        dx2 = jnp.where(keep, dx2, 0.0)

        dy_corr = 0.5 * (gx * dx0 + gy * dx1 + gs * dx2)
        y_ref[0, 0, d] = z0 + dy_corr + jnp.where(mask, STRICT_BONUS, 0.0)

        coords_ref[0, 0, 0, d] = float(d) + dx2
        coords_ref[0, 0, 1, d] = row_f + dx1
        coords_ref[0, 0, 2, d] = col_f + dx0


@jax.jit
def kernel(x):
    B, C, D, H, W = x.shape
    noise = jnp.abs(jax.random.uniform(jax.random.key(42), (3, 3), dtype=x.dtype)) * NOISE_EPS
    xr = x.reshape(B * C, D, H, W)
    coords, y = pl.pallas_call(
        functools.partial(_stencil_kernel, D=D, H=H, W=W),
        grid=(B * C,),
        out_shape=(
            jax.ShapeDtypeStruct((B, C, 3, D, H, W), x.dtype),
            jax.ShapeDtypeStruct((B, C, D, H, W), x.dtype),
        ),
        in_specs=[
            pl.BlockSpec((1, D, H, W), lambda b: (b, 0, 0, 0)),
            pl.BlockSpec(memory_space=pltpu.SMEM),
        ],
        out_specs=(
            pl.BlockSpec((1, 1, 3, D, H, W), lambda b: (b, 0, 0, 0, 0, 0)),
            pl.BlockSpec((1, 1, D, H, W), lambda b: (b, 0, 0, 0, 0)),
        ),
        compiler_params=pltpu.CompilerParams(
            dimension_semantics=("parallel",),
        ),
    )(xr, noise)
    return coords, y
